# Initial kernel scaffold; baseline (speedup 1.0000x reference)
import jax
import jax.numpy as jnp
from jax.experimental import pallas as pl


def kernel(point_coordinates, point_attributes):
    raise NotImplementedError("write your pallas kernel here")


# speedup vs baseline: 4.9125x; 4.9125x over previous
import functools

import jax
import jax.numpy as jnp
from jax import lax
from jax.experimental import pallas as pl
from jax.experimental.pallas import tpu as pltpu
from jax.experimental.pallas import tpu_sc as plsc

B = 2
C = 8
N = 262144
NT = B * N
GW, GL, GH = 128, 128, 64
NV = GW * GL * GH
SLAB = GL * GH
NBINS = B * GW
NBINS_PAD = 272
SENTINEL_KEY = B * NV
MIN_PTS = 10
VS = 0.4
OFF = 0.5
OX = -25.6
OY = -25.6
OZ = -12.8

NWORKERS = 32
PPT = NT // NWORKERS
CH = 1024
NCH = PPT // CH
RECW = 16
BPT = NBINS // NWORKERS
REC_ROWS = NT + CH

_mesh = plsc.VectorSubcoreMesh(core_axis_name="c", subcore_axis_name="s")
_CP = pltpu.CompilerParams(needs_layout_passes=False, use_tc_tiling_on_sc=False)


def _wid():
    return lax.axis_index("s") * 2 + lax.axis_index("c")


@functools.partial(
    pl.kernel,
    out_type=(
        jax.ShapeDtypeStruct((NT,), jnp.int32),
        jax.ShapeDtypeStruct((NWORKERS, NBINS_PAD), jnp.int32),
    ),
    mesh=_mesh,
    compiler_params=_CP,
    scratch_types=[
        pltpu.VMEM((3, CH), jnp.float32),
        pltpu.VMEM((CH,), jnp.int32),
        pltpu.VMEM((NBINS_PAD,), jnp.int32),
    ],
)
def _k1(coords_hbm, keys_hbm, hists_hbm, cstage, kstage, hist):
    wid = _wid()
    b = wid // 16
    pstart0 = (wid % 16) * PPT
    zi16 = jnp.zeros((16,), jnp.int32)

    def zero_hist(i, _):
        hist[pl.ds(i * 16, 16)] = zi16
        return 0

    lax.fori_loop(0, NBINS_PAD // 16, zero_hist, 0)

    ones = jnp.ones((16,), jnp.int32)

    def do_chunk(ch, _):
        pstart = pstart0 + ch * CH
        pltpu.sync_copy(coords_hbm.at[b, :, pl.ds(pstart, CH)], cstage)

        def do_group(g, _):
            s = g * 16
            px = cstage[0, pl.ds(s, 16)]
            py = cstage[1, pl.ds(s, 16)]
            pz = cstage[2, pl.ds(s, 16)]
            xi = ((px - OX) / VS + OFF).astype(jnp.int32)
            yi = ((py - OY) / VS + OFF).astype(jnp.int32)
            zi = ((pz - OZ) / VS + OFF).astype(jnp.int32)
            valid = (xi < GW) & (yi < GL) & (zi < GH)
            key = (xi << 13) + (yi << 6) + zi + b * NV
            key = jnp.where(valid, key, SENTINEL_KEY)
            kstage[pl.ds(s, 16)] = key
            plsc.addupdate_scatter(hist, [key >> 13], ones)
            return 0

        lax.fori_loop(0, CH // 16, do_group, 0)
        pltpu.sync_copy(kstage, keys_hbm.at[pl.ds(b * N + pstart, CH)])
        return 0

    lax.fori_loop(0, NCH, do_chunk, 0)
    pltpu.sync_copy(hist, hists_hbm.at[wid])


@functools.partial(
    pl.kernel,
    out_type=jax.ShapeDtypeStruct((REC_ROWS, RECW), jnp.float32),
    mesh=_mesh,
    compiler_params=_CP,
    scratch_types=[
        pltpu.VMEM((NWORKERS, NBINS_PAD), jnp.int32),
        pltpu.VMEM((NBINS_PAD,), jnp.int32),
        pltpu.VMEM((CH,), jnp.int32),
        pltpu.VMEM((C, CH), jnp.float32),
        pltpu.VMEM((CH, RECW), jnp.float32),
        pltpu.VMEM((CH,), jnp.int32),
        pltpu.SemaphoreType.DMA,
    ],
)
def _k2(keys_hbm, attrs_hbm, hists_hbm, rec_hbm,
        hists_v, off, kstage, astage, rec, addr, sem):
    wid = _wid()
    b = wid // 16
    pstart0 = (wid % 16) * PPT
    widv = jnp.full((16,), wid, jnp.int32)
    lanes = lax.iota(jnp.int32, 16)

    pltpu.sync_copy(hists_hbm, hists_v)

    carry = jnp.int32(0)
    for cki in range(NBINS_PAD // 16):
        sl = pl.ds(cki * 16, 16)

        def acc_t(t, tb):
            tot, below = tb
            h = hists_v[t, sl]
            tot = tot + h
            below = below + jnp.where(jnp.full((16,), t, jnp.int32) < widv, h,
                                      jnp.zeros((16,), jnp.int32))
            return (tot, below)

        tot, below = lax.fori_loop(
            0, NWORKERS, acc_t,
            (jnp.zeros((16,), jnp.int32), jnp.zeros((16,), jnp.int32)))
        excl = plsc.cumsum(tot) - tot + carry
        off[sl] = excl + below
        carry = carry + jnp.sum(tot)

    def do_chunk(ch, _):
        pstart = pstart0 + ch * CH
        pltpu.sync_copy(keys_hbm.at[pl.ds(b * N + pstart, CH)], kstage)
        pltpu.sync_copy(attrs_hbm.at[b, :, pl.ds(pstart, CH)], astage)

        def do_group(g, _):
            s = g * 16
            key = kstage[pl.ds(s, 16)]
            bin_ = key >> 13
            cnt, last = plsc.scan_count(bin_)
            ofs = plsc.load_gather(off, [bin_])
            addr[pl.ds(s, 16)] = ofs + cnt - 1
            plsc.addupdate_scatter(off, [bin_], cnt, mask=last)
            rows = s + lanes
            plsc.store_scatter(rec, [rows, jnp.zeros((16,), jnp.int32)],
                               plsc.bitcast(key, jnp.float32))
            for c in range(C):
                a = astage[c, pl.ds(s, 16)]
                plsc.store_scatter(rec, [rows, jnp.full((16,), c + 1, jnp.int32)], a)
            return 0

        lax.fori_loop(0, CH // 16, do_group, 0)
        pltpu.async_copy(rec, rec_hbm.at[addr], sem).wait()
        return 0

    lax.fori_loop(0, NCH, do_chunk, 0)


@functools.partial(
    pl.kernel,
    out_type=(
        jax.ShapeDtypeStruct((B, C, GW, SLAB), jnp.float32),
        jax.ShapeDtypeStruct((B, GW, SLAB), jnp.float32),
    ),
    mesh=_mesh,
    compiler_params=_CP,
    scratch_types=[
        pltpu.VMEM((NWORKERS, NBINS_PAD), jnp.int32),
        pltpu.VMEM((NBINS_PAD,), jnp.int32),
        pltpu.VMEM((NBINS_PAD,), jnp.int32),
        pltpu.VMEM((CH, RECW), jnp.float32),
        pltpu.VMEM((C, SLAB), jnp.float32),
        pltpu.VMEM((SLAB,), jnp.int32),
        pltpu.VMEM((SLAB,), jnp.float32),
        pltpu.SemaphoreType.DMA,
    ],
)
def _k3(rec_hbm, hists_hbm, vox_hbm, occ_hbm,
        hists_v, bstart, bcnt, rstage, vacc, cntv, occf, sem):
    wid = _wid()
    lanes = lax.iota(jnp.int32, 16)
    zf16 = jnp.zeros((16,), jnp.float32)
    zi16 = jnp.zeros((16,), jnp.int32)
    ones = jnp.ones((16,), jnp.int32)

    pltpu.sync_copy(hists_hbm, hists_v)

    carry = jnp.int32(0)
    for cki in range(NBINS_PAD // 16):
        sl = pl.ds(cki * 16, 16)

        def acc_t(t, tot):
            return tot + hists_v[t, sl]

        tot = lax.fori_loop(0, NWORKERS, acc_t, jnp.zeros((16,), jnp.int32))
        bcnt[sl] = tot
        bstart[sl] = plsc.cumsum(tot) - tot + carry
        carry = carry + jnp.sum(tot)

    def do_bin(p, _):
        bin_ = wid * BPT + p
        binv = jnp.full((16,), bin_, jnp.int32)
        start = plsc.load_gather(bstart, [binv])[0]
        count = plsc.load_gather(bcnt, [binv])[0]

        def zeroi(i, _):
            sl = pl.ds(i * 16, 16)
            for c in range(C):
                vacc[c, sl] = zf16
            cntv[sl] = zi16
            return 0

        lax.fori_loop(0, SLAB // 16, zeroi, 0)

        nchunks = (count + CH - 1) // CH

        def do_chunk(ch, _):
            pltpu.sync_copy(rec_hbm.at[pl.ds(start + ch * CH, CH), :], rstage)
            m = jnp.minimum(count - ch * CH, CH)

            def do_group(g, _):
                s = g * 16
                pos = s + lanes
                valid = pos < m
                kf = plsc.load_gather(rstage, [pos, zi16])
                key = plsc.bitcast(kf, jnp.int32)
                local = key & (SLAB - 1)
                cnt, _last = plsc.scan_count(local, mask=valid)
                rank = cnt - 1
                plsc.addupdate_scatter(cntv, [local], ones, mask=valid)
                maxrank = jnp.max(jnp.where(valid, rank, 0))
                vals = [plsc.load_gather(rstage,
                                         [pos, jnp.full((16,), c + 1, jnp.int32)])
                        for c in range(C)]

                def round_cond(r):
                    return r <= maxrank

                def round_body(r):
                    active = valid & (rank == r)
                    for c in range(C):
                        old = plsc.load_gather(vacc.at[c], [local])
                        plsc.store_scatter(vacc.at[c], [local],
                                           jnp.maximum(old, vals[c]),
                                           mask=active)
                    return r + 1

                lax.while_loop(round_cond, round_body, jnp.int32(0))
                return 0

            lax.fori_loop(0, (m + 15) // 16, do_group, 0)
            return 0

        lax.fori_loop(0, nchunks, do_chunk, 0)

        ten = jnp.full((16,), MIN_PTS, jnp.int32)
        onef = jnp.ones((16,), jnp.float32)

        def gate(i, _):
            sl = pl.ds(i * 16, 16)
            occ = jnp.where(cntv[sl] >= ten, onef, zf16)
            occf[sl] = occ
            for c in range(C):
                vacc[c, sl] = vacc[c, sl] * occ
            return 0

        lax.fori_loop(0, SLAB // 16, gate, 0)

        b = bin_ >> 7
        x = bin_ & (GW - 1)
        cps = [pltpu.async_copy(vacc.at[c], vox_hbm.at[b, c, x], sem)
               for c in range(C)]
        cps.append(pltpu.async_copy(occf, occ_hbm.at[b, x], sem))
        for cp in cps:
            cp.wait()
        return 0

    lax.fori_loop(0, BPT, do_bin, 0)


def kernel(point_coordinates, point_attributes):
    coords = point_coordinates.reshape(B, 3, N)
    attrs = point_attributes.astype(jnp.float32).reshape(B, C, N)
    keys, hists = _k1(coords)
    recs = _k2(keys, attrs, hists)
    vox, occ = _k3(recs, hists)
    return (vox.reshape(B, C, GW, GL, GH),
            occ.reshape(B, 1, GW, GL, GH))
